# baseline (speedup 1.0000x reference)
import functools

import jax
import jax.numpy as jnp
from jax import lax
from jax.experimental import pallas as pl
from jax.experimental.pallas import tpu as pltpu
from jax.experimental.pallas import tpu_sc as plsc

N = 10000
E = 320000
D = 128
H = D // 2
D2 = 2 * D
EPS = 1e-7

NC = 2
NS = 16
CH = 128
ZR = 78
EPT = E // NS
NCH = EPT // CH
TAIL = EPT - NCH * CH
RPT = (N // NS) // 8 * 8
RTAIL = N - RPT * NS

BN_BLK = 1000
GRID = N // BN_BLK


def _sc_body(ep_hbm, si_hbm, di_hbm, out_hbm, acc,
             ia0, ia1, ia2, id0, id1, id2, bf0, bf1, bf2,
             ibt, idt,
             is0, is1, is2, gs0, gs1, gs2, tsem):
    c = lax.axis_index("c")
    s = lax.axis_index("s")
    base = s * EPT
    ibs = [ia0, ia1, ia2]
    ibd = [id0, id1, id2]
    buf = [bf0, bf1, bf2]
    isem = [is0, is1, is2]
    gsem = [gs0, gs1, gs2]

    def idx_start(chunk, slot):
        off = base + chunk * CH
        pltpu.async_copy(si_hbm.at[pl.ds(c * E + off, CH)], ibs[slot],
                         isem[slot])
        pltpu.async_copy(di_hbm.at[pl.ds(off, CH)], ibd[slot], isem[slot])

    def idx_wait(slot):
        pltpu.make_async_copy(si_hbm.at[pl.ds(0, CH)], ibs[slot],
                              isem[slot]).wait()
        pltpu.make_async_copy(di_hbm.at[pl.ds(0, CH)], ibd[slot],
                              isem[slot]).wait()

    def gather_start(slot):
        pltpu.async_copy(ep_hbm.at[ibs[slot]], buf[slot], gsem[slot])

    def gather_wait(slot):
        pltpu.make_async_copy(ep_hbm.at[ibs[slot]], buf[slot],
                              gsem[slot]).wait()

    def scatter_sync(slot):
        pltpu.sync_copy(buf[slot], acc.at[ibd[slot]], add=True)

    idx_start(0, 0)
    idx_start(1, 1)
    idx_start(2, 2)

    def _z(i, _):
        bf2[i // 8, pl.ds((i % 8) * 16, 16)] = jnp.zeros((16,), jnp.float32)
        return 0
    lax.fori_loop(0, ZR * 8, _z, 0)

    def _zcp(j, _):
        pltpu.sync_copy(bf2.at[pl.ds(0, ZR)],
                        acc.at[pl.ds(s * RPT + j * ZR, ZR)])
        return 0
    lax.fori_loop(0, RPT // ZR, _zcp, 0)
    if RTAIL:
        @pl.when(s == 0)
        def _():
            pltpu.sync_copy(bf2.at[pl.ds(0, RTAIL)],
                            acc.at[pl.ds(NS * RPT, RTAIL)])

    idx_wait(0)
    gather_start(0)
    idx_wait(1)
    gather_start(1)
    plsc.subcore_barrier()

    def _chunk_body(chk, slot, pf_gather, pf_idx):
        gather_wait(slot)
        if pf_gather:
            idx_wait((slot + 2) % 3)
            gather_start((slot + 2) % 3)
        scatter_sync(slot)
        if pf_idx:
            idx_start(chk + 3, slot)

    def _loop(m, _):
        for k in range(3):
            _chunk_body(3 * m + k, k, True, True)
        return 0
    lax.fori_loop(0, (NCH - 3) // 3, _loop, 0)

    _chunk_body(NCH - 3, 0, True, False)
    _chunk_body(NCH - 2, 1, False, False)
    _chunk_body(NCH - 1, 2, False, False)

    if TAIL:
        toff = base + NCH * CH
        pltpu.sync_copy(si_hbm.at[pl.ds(c * E + toff, TAIL)], ibt)
        pltpu.sync_copy(di_hbm.at[pl.ds(toff, TAIL)], idt)
        pltpu.async_copy(ep_hbm.at[ibt], bf0.at[pl.ds(0, TAIL)], tsem).wait()
        pltpu.sync_copy(bf0.at[pl.ds(0, TAIL)], acc.at[idt], add=True)

    plsc.subcore_barrier()
    pltpu.sync_copy(acc.at[pl.ds(s * RPT, RPT)],
                    out_hbm.at[pl.ds(c * N + s * RPT, RPT)])
    if RTAIL:
        @pl.when(s == 0)
        def _():
            pltpu.sync_copy(acc.at[pl.ds(NS * RPT, RTAIL)],
                            out_hbm.at[pl.ds(c * N + NS * RPT, RTAIL)])


@jax.jit
def _sc_agg(ep_flat, sidx, didx):
    mesh = plsc.VectorSubcoreMesh(core_axis_name="c", subcore_axis_name="s")
    f = pl.kernel(
        _sc_body,
        out_type=jax.ShapeDtypeStruct((2 * N, D), jnp.float32),
        mesh=mesh,
        scratch_types=(
            [pltpu.VMEM_SHARED((N, D), jnp.float32)]
            + [pltpu.VMEM((CH,), jnp.int32)] * 6
            + [pltpu.VMEM((CH, D), jnp.float32)] * 3
            + [pltpu.VMEM((max(TAIL, 16),), jnp.int32)] * 2
            + [pltpu.SemaphoreType.DMA] * 7
        ),
    )
    return f(ep_flat, sidx, didx)


def _prep_body(v_ref, t_ref, ep_ref):
    m = jnp.maximum(v_ref[...], 0.0) + EPS
    e = jnp.exp(m * t_ref[0, 0])
    p = m * e
    ep_ref[0] = jnp.concatenate([e[:, :H], p[:, :H]], axis=1)
    ep_ref[1] = jnp.concatenate([e[:, H:], p[:, H:]], axis=1)


@jax.jit
def _prep(v, t):
    return pl.pallas_call(
        _prep_body,
        grid=(GRID,),
        in_specs=[
            pl.BlockSpec((BN_BLK, D), lambda i: (i, 0)),
            pl.BlockSpec(memory_space=pltpu.SMEM),
        ],
        out_specs=pl.BlockSpec((2, BN_BLK, D), lambda i: (0, i, 0)),
        out_shape=jax.ShapeDtypeStruct((2, N, D), jnp.float32),
    )(v, t.reshape(1, 1))


def _stage_a_body(acc_ref, v_ref, w1t_ref, b1_ref, sc_ref, h1_ref, st_ref):
    i = pl.program_id(0)
    den = jnp.concatenate([acc_ref[0, :, :H], acc_ref[1, :, :H]], axis=1)
    num = jnp.concatenate([acc_ref[0, :, H:], acc_ref[1, :, H:]], axis=1)
    out = num / (den + 1e-16)
    v = v_ref[...]
    nrm = jnp.maximum(
        jnp.sqrt(jnp.sum(out * out, axis=1, keepdims=True)), 1e-12)
    xn = jnp.sqrt(jnp.sum(v * v, axis=1, keepdims=True))
    z = out * (xn / nrm * sc_ref[0, 0]) + v
    h1 = jnp.dot(z, w1t_ref[...], preferred_element_type=jnp.float32) \
        + b1_ref[...]
    h1_ref[...] = h1
    st = jnp.concatenate([jnp.sum(h1, axis=0, keepdims=True),
                          jnp.sum(h1 * h1, axis=0, keepdims=True)], axis=0)

    @pl.when(i == 0)
    def _():
        st_ref[...] = st

    @pl.when(i > 0)
    def _():
        st_ref[...] = st_ref[...] + st


@jax.jit
def _stage_a(acc, v, w1t, b1, scale):
    return pl.pallas_call(
        _stage_a_body,
        grid=(GRID,),
        in_specs=[
            pl.BlockSpec((2, BN_BLK, D), lambda i: (0, i, 0)),
            pl.BlockSpec((BN_BLK, D), lambda i: (i, 0)),
            pl.BlockSpec((D, D2), lambda i: (0, 0)),
            pl.BlockSpec((1, D2), lambda i: (0, 0)),
            pl.BlockSpec(memory_space=pltpu.SMEM),
        ],
        out_specs=[
            pl.BlockSpec((BN_BLK, D2), lambda i: (i, 0)),
            pl.BlockSpec((2, D2), lambda i: (0, 0)),
        ],
        out_shape=[
            jax.ShapeDtypeStruct((N, D2), jnp.float32),
            jax.ShapeDtypeStruct((2, D2), jnp.float32),
        ],
    )(acc, v, w1t, b1.reshape(1, D2), scale.reshape(1, 1))


def _stage_b_body(h1_ref, bn_ref, w2t_ref, b2_ref, h2_ref, st_ref):
    i = pl.program_id(0)
    a = (h1_ref[...] - bn_ref[0:1, :]) * bn_ref[1:2, :] * bn_ref[2:3, :] \
        + bn_ref[3:4, :]
    a = jnp.maximum(a, 0.0)
    h2 = jnp.dot(a, w2t_ref[...], preferred_element_type=jnp.float32) \
        + b2_ref[...]
    h2_ref[...] = h2
    st = jnp.concatenate([jnp.sum(h2, axis=0, keepdims=True),
                          jnp.sum(h2 * h2, axis=0, keepdims=True)], axis=0)

    @pl.when(i == 0)
    def _():
        st_ref[...] = st

    @pl.when(i > 0)
    def _():
        st_ref[...] = st_ref[...] + st


@jax.jit
def _stage_b(h1, mu1, inv1, g1, be1, w2t, b2):
    bn = jnp.stack([mu1, inv1, g1, be1], axis=0)
    return pl.pallas_call(
        _stage_b_body,
        grid=(GRID,),
        in_specs=[
            pl.BlockSpec((BN_BLK, D2), lambda i: (i, 0)),
            pl.BlockSpec((4, D2), lambda i: (0, 0)),
            pl.BlockSpec((D2, D), lambda i: (0, 0)),
            pl.BlockSpec((1, D), lambda i: (0, 0)),
        ],
        out_specs=[
            pl.BlockSpec((BN_BLK, D), lambda i: (i, 0)),
            pl.BlockSpec((2, D), lambda i: (0, 0)),
        ],
        out_shape=[
            jax.ShapeDtypeStruct((N, D), jnp.float32),
            jax.ShapeDtypeStruct((2, D), jnp.float32),
        ],
    )(h1, bn, w2t, b2.reshape(1, D))


def _stage_c0_body(h2_ref, bn_ref, t_ref, x2_ref, ep_ref):
    y = (h2_ref[...] - bn_ref[0:1, :]) * bn_ref[1:2, :] * bn_ref[2:3, :] \
        + bn_ref[3:4, :]
    x2 = jnp.maximum(y, 0.0) + EPS
    x2_ref[...] = x2
    m = jnp.maximum(x2, 0.0) + EPS
    e = jnp.exp(m * t_ref[0, 0])
    p = m * e
    ep_ref[0] = jnp.concatenate([e[:, :H], p[:, :H]], axis=1)
    ep_ref[1] = jnp.concatenate([e[:, H:], p[:, H:]], axis=1)


@jax.jit
def _stage_c0(h2, mu2, inv2, g, b, t_next):
    bn = jnp.stack([mu2, inv2, g, b], axis=0)
    return pl.pallas_call(
        _stage_c0_body,
        grid=(GRID,),
        in_specs=[
            pl.BlockSpec((BN_BLK, D), lambda i: (i, 0)),
            pl.BlockSpec((4, D), lambda i: (0, 0)),
            pl.BlockSpec(memory_space=pltpu.SMEM),
        ],
        out_specs=[
            pl.BlockSpec((BN_BLK, D), lambda i: (i, 0)),
            pl.BlockSpec((2, BN_BLK, D), lambda i: (0, i, 0)),
        ],
        out_shape=[
            jax.ShapeDtypeStruct((N, D), jnp.float32),
            jax.ShapeDtypeStruct((2, N, D), jnp.float32),
        ],
    )(h2, bn, t_next.reshape(1, 1))


def _stage_c1_body(h2_ref, bn_ref, x_ref, out_ref):
    y = (h2_ref[...] - bn_ref[0:1, :]) * bn_ref[1:2, :] * bn_ref[2:3, :] \
        + bn_ref[3:4, :]
    out_ref[...] = jnp.maximum(x_ref[...] + y, 0.0) + EPS


@jax.jit
def _stage_c1(h2, mu2, inv2, g, b, x0):
    bn = jnp.stack([mu2, inv2, g, b], axis=0)
    return pl.pallas_call(
        _stage_c1_body,
        grid=(GRID,),
        in_specs=[
            pl.BlockSpec((BN_BLK, D), lambda i: (i, 0)),
            pl.BlockSpec((4, D), lambda i: (0, 0)),
            pl.BlockSpec((BN_BLK, D), lambda i: (i, 0)),
        ],
        out_specs=pl.BlockSpec((BN_BLK, D), lambda i: (i, 0)),
        out_shape=jax.ShapeDtypeStruct((N, D), jnp.float32),
    )(h2, bn, x0)


def _finalize_stats(st):
    mu = st[0] / N
    var = st[1] / N - mu * mu
    return mu, lax.rsqrt(var + 1e-5)


def kernel(x, edge_index, params):
    src = edge_index[0]
    dst = edge_index[1]
    sidx = jnp.concatenate([src, src + N])
    convs = params["convs"]

    v = x
    ep = _prep(v, convs[0]["t"]).reshape(2 * N, D)
    for i in range(2):
        p = convs[i]
        acc = _sc_agg(ep, sidx, dst)
        h1, st1 = _stage_a(acc.reshape(2, N, D), v, p["W1"].T, p["b1"],
                           p["scale"])
        mu1, inv1 = _finalize_stats(st1)
        h2, st2 = _stage_b(h1, mu1, inv1, p["g1"], p["be1"], p["W2"].T,
                           p["b2"])
        mu2, inv2 = _finalize_stats(st2)
        if i == 0:
            v, ep2 = _stage_c0(h2, mu2, inv2, params["norm_g"][0],
                               params["norm_b"][0], convs[1]["t"])
            ep = ep2.reshape(2 * N, D)
        else:
            out = _stage_c1(h2, mu2, inv2, params["norm_g"][1],
                            params["norm_b"][1], x)
    return out


# speedup vs baseline: 1.0576x; 1.0576x over previous
import functools

import jax
import jax.numpy as jnp
from jax import lax
from jax.experimental import pallas as pl
from jax.experimental.pallas import tpu as pltpu
from jax.experimental.pallas import tpu_sc as plsc

N = 10000
E = 320000
D = 128
H = D // 2
D2 = 2 * D
EPS = 1e-7

NC = 2
NS = 16
CH = 104
EPT = E // NS
NCH = EPT // CH
TAIL = EPT - NCH * CH
RPT = (N // NS) // 8 * 8
RTAIL = N - RPT * NS

BN_BLK = 1000
GRID = N // BN_BLK


def _sc_body(ep_hbm, si_hbm, di_hbm, out_hbm, acc,
             ia0, ia1, ia2, id0, id1, id2, bf0, bf1, bf2,
             ibt, idt,
             is0, is1, is2, gs0, gs1, gs2, tsem):
    c = lax.axis_index("c")
    s = lax.axis_index("s")
    base = s * EPT
    ibs = [ia0, ia1, ia2]
    ibd = [id0, id1, id2]
    buf = [bf0, bf1, bf2]
    isem = [is0, is1, is2]
    gsem = [gs0, gs1, gs2]

    def idx_start(chunk, slot):
        off = base + chunk * CH
        pltpu.async_copy(si_hbm.at[pl.ds(c * E + off, CH)], ibs[slot],
                         isem[slot])
        pltpu.async_copy(di_hbm.at[pl.ds(off, CH)], ibd[slot], isem[slot])

    def idx_wait(slot):
        pltpu.make_async_copy(si_hbm.at[pl.ds(0, CH)], ibs[slot],
                              isem[slot]).wait()
        pltpu.make_async_copy(di_hbm.at[pl.ds(0, CH)], ibd[slot],
                              isem[slot]).wait()

    def gather_start(slot):
        pltpu.async_copy(ep_hbm.at[ibs[slot]], buf[slot], gsem[slot])

    def gather_wait(slot):
        pltpu.make_async_copy(ep_hbm.at[ibs[slot]], buf[slot],
                              gsem[slot]).wait()

    def scatter_sync(slot):
        pltpu.sync_copy(buf[slot], acc.at[ibd[slot]], add=True)

    idx_start(0, 0)
    idx_start(1, 1)
    idx_start(2, 2)

    def _z(i, _):
        bf2[i // 8, pl.ds((i % 8) * 16, 16)] = jnp.zeros((16,), jnp.float32)
        return 0
    lax.fori_loop(0, CH * 8, _z, 0)

    def _zcp(j, _):
        pltpu.sync_copy(bf2, acc.at[pl.ds(s * RPT + j * CH, CH)])
        return 0
    lax.fori_loop(0, RPT // CH, _zcp, 0)
    if RTAIL:
        @pl.when(s == 0)
        def _():
            pltpu.sync_copy(bf2.at[pl.ds(0, RTAIL)],
                            acc.at[pl.ds(NS * RPT, RTAIL)])

    idx_wait(0)
    gather_start(0)
    idx_wait(1)
    gather_start(1)
    plsc.subcore_barrier()

    def _chunk_body(chk, slot, prefetch):
        gather_wait(slot)
        scatter_sync(slot)
        if prefetch:
            idx_start(chk + 3, slot)
            idx_wait((slot + 2) % 3)
            gather_start((slot + 2) % 3)

    def _loop(m, _):
        for k in range(3):
            _chunk_body(3 * m + k, k, True)
        return 0
    lax.fori_loop(0, (NCH - 3) // 3, _loop, 0)

    _chunk_body(NCH - 3, 0, False)
    idx_wait(2)
    gather_start(2)
    _chunk_body(NCH - 2, 1, False)
    _chunk_body(NCH - 1, 2, False)

    if TAIL:
        toff = base + NCH * CH
        pltpu.sync_copy(si_hbm.at[pl.ds(c * E + toff, TAIL)], ibt)
        pltpu.sync_copy(di_hbm.at[pl.ds(toff, TAIL)], idt)
        pltpu.async_copy(ep_hbm.at[ibt], bf0.at[pl.ds(0, TAIL)], tsem).wait()
        pltpu.sync_copy(bf0.at[pl.ds(0, TAIL)], acc.at[idt], add=True)

    plsc.subcore_barrier()
    pltpu.sync_copy(acc.at[pl.ds(s * RPT, RPT)],
                    out_hbm.at[pl.ds(c * N + s * RPT, RPT)])
    if RTAIL:
        @pl.when(s == 0)
        def _():
            pltpu.sync_copy(acc.at[pl.ds(NS * RPT, RTAIL)],
                            out_hbm.at[pl.ds(c * N + NS * RPT, RTAIL)])


@jax.jit
def _sc_agg(ep_flat, sidx, didx):
    mesh = plsc.VectorSubcoreMesh(core_axis_name="c", subcore_axis_name="s")
    f = pl.kernel(
        _sc_body,
        out_type=jax.ShapeDtypeStruct((2 * N, D), jnp.float32),
        mesh=mesh,
        scratch_types=(
            [pltpu.VMEM_SHARED((N, D), jnp.float32)]
            + [pltpu.VMEM((CH,), jnp.int32)] * 6
            + [pltpu.VMEM((CH, D), jnp.float32)] * 3
            + [pltpu.VMEM((max(TAIL, 16),), jnp.int32)] * 2
            + [pltpu.SemaphoreType.DMA] * 7
        ),
    )
    return f(ep_flat, sidx, didx)


def _prep_body(v_ref, t_ref, ep_ref):
    m = jnp.maximum(v_ref[...], 0.0) + EPS
    e = jnp.exp(m * t_ref[0, 0])
    p = m * e
    ep_ref[0] = jnp.concatenate([e[:, :H], p[:, :H]], axis=1)
    ep_ref[1] = jnp.concatenate([e[:, H:], p[:, H:]], axis=1)


@jax.jit
def _prep(v, t):
    return pl.pallas_call(
        _prep_body,
        grid=(GRID,),
        in_specs=[
            pl.BlockSpec((BN_BLK, D), lambda i: (i, 0)),
            pl.BlockSpec(memory_space=pltpu.SMEM),
        ],
        out_specs=pl.BlockSpec((2, BN_BLK, D), lambda i: (0, i, 0)),
        out_shape=jax.ShapeDtypeStruct((2, N, D), jnp.float32),
    )(v, t.reshape(1, 1))


def _stage_a_body(acc_ref, v_ref, w1t_ref, b1_ref, sc_ref, h1_ref, st_ref):
    i = pl.program_id(0)
    den = jnp.concatenate([acc_ref[0, :, :H], acc_ref[1, :, :H]], axis=1)
    num = jnp.concatenate([acc_ref[0, :, H:], acc_ref[1, :, H:]], axis=1)
    out = num / (den + 1e-16)
    v = v_ref[...]
    nrm = jnp.maximum(
        jnp.sqrt(jnp.sum(out * out, axis=1, keepdims=True)), 1e-12)
    xn = jnp.sqrt(jnp.sum(v * v, axis=1, keepdims=True))
    z = out * (xn / nrm * sc_ref[0, 0]) + v
    h1 = jnp.dot(z, w1t_ref[...], preferred_element_type=jnp.float32) \
        + b1_ref[...]
    h1_ref[...] = h1
    st = jnp.concatenate([jnp.sum(h1, axis=0, keepdims=True),
                          jnp.sum(h1 * h1, axis=0, keepdims=True)], axis=0)

    @pl.when(i == 0)
    def _():
        st_ref[...] = st

    @pl.when(i > 0)
    def _():
        st_ref[...] = st_ref[...] + st


@jax.jit
def _stage_a(acc, v, w1t, b1, scale):
    return pl.pallas_call(
        _stage_a_body,
        grid=(GRID,),
        in_specs=[
            pl.BlockSpec((2, BN_BLK, D), lambda i: (0, i, 0)),
            pl.BlockSpec((BN_BLK, D), lambda i: (i, 0)),
            pl.BlockSpec((D, D2), lambda i: (0, 0)),
            pl.BlockSpec((1, D2), lambda i: (0, 0)),
            pl.BlockSpec(memory_space=pltpu.SMEM),
        ],
        out_specs=[
            pl.BlockSpec((BN_BLK, D2), lambda i: (i, 0)),
            pl.BlockSpec((2, D2), lambda i: (0, 0)),
        ],
        out_shape=[
            jax.ShapeDtypeStruct((N, D2), jnp.float32),
            jax.ShapeDtypeStruct((2, D2), jnp.float32),
        ],
    )(acc, v, w1t, b1.reshape(1, D2), scale.reshape(1, 1))


def _stage_b_body(h1_ref, bn_ref, w2t_ref, b2_ref, h2_ref, st_ref):
    i = pl.program_id(0)
    a = (h1_ref[...] - bn_ref[0:1, :]) * bn_ref[1:2, :] * bn_ref[2:3, :] \
        + bn_ref[3:4, :]
    a = jnp.maximum(a, 0.0)
    h2 = jnp.dot(a, w2t_ref[...], preferred_element_type=jnp.float32) \
        + b2_ref[...]
    h2_ref[...] = h2
    st = jnp.concatenate([jnp.sum(h2, axis=0, keepdims=True),
                          jnp.sum(h2 * h2, axis=0, keepdims=True)], axis=0)

    @pl.when(i == 0)
    def _():
        st_ref[...] = st

    @pl.when(i > 0)
    def _():
        st_ref[...] = st_ref[...] + st


@jax.jit
def _stage_b(h1, mu1, inv1, g1, be1, w2t, b2):
    bn = jnp.stack([mu1, inv1, g1, be1], axis=0)
    return pl.pallas_call(
        _stage_b_body,
        grid=(GRID,),
        in_specs=[
            pl.BlockSpec((BN_BLK, D2), lambda i: (i, 0)),
            pl.BlockSpec((4, D2), lambda i: (0, 0)),
            pl.BlockSpec((D2, D), lambda i: (0, 0)),
            pl.BlockSpec((1, D), lambda i: (0, 0)),
        ],
        out_specs=[
            pl.BlockSpec((BN_BLK, D), lambda i: (i, 0)),
            pl.BlockSpec((2, D), lambda i: (0, 0)),
        ],
        out_shape=[
            jax.ShapeDtypeStruct((N, D), jnp.float32),
            jax.ShapeDtypeStruct((2, D), jnp.float32),
        ],
    )(h1, bn, w2t, b2.reshape(1, D))


def _stage_c0_body(h2_ref, bn_ref, t_ref, x2_ref, ep_ref):
    y = (h2_ref[...] - bn_ref[0:1, :]) * bn_ref[1:2, :] * bn_ref[2:3, :] \
        + bn_ref[3:4, :]
    x2 = jnp.maximum(y, 0.0) + EPS
    x2_ref[...] = x2
    m = jnp.maximum(x2, 0.0) + EPS
    e = jnp.exp(m * t_ref[0, 0])
    p = m * e
    ep_ref[0] = jnp.concatenate([e[:, :H], p[:, :H]], axis=1)
    ep_ref[1] = jnp.concatenate([e[:, H:], p[:, H:]], axis=1)


@jax.jit
def _stage_c0(h2, mu2, inv2, g, b, t_next):
    bn = jnp.stack([mu2, inv2, g, b], axis=0)
    return pl.pallas_call(
        _stage_c0_body,
        grid=(GRID,),
        in_specs=[
            pl.BlockSpec((BN_BLK, D), lambda i: (i, 0)),
            pl.BlockSpec((4, D), lambda i: (0, 0)),
            pl.BlockSpec(memory_space=pltpu.SMEM),
        ],
        out_specs=[
            pl.BlockSpec((BN_BLK, D), lambda i: (i, 0)),
            pl.BlockSpec((2, BN_BLK, D), lambda i: (0, i, 0)),
        ],
        out_shape=[
            jax.ShapeDtypeStruct((N, D), jnp.float32),
            jax.ShapeDtypeStruct((2, N, D), jnp.float32),
        ],
    )(h2, bn, t_next.reshape(1, 1))


def _stage_c1_body(h2_ref, bn_ref, x_ref, out_ref):
    y = (h2_ref[...] - bn_ref[0:1, :]) * bn_ref[1:2, :] * bn_ref[2:3, :] \
        + bn_ref[3:4, :]
    out_ref[...] = jnp.maximum(x_ref[...] + y, 0.0) + EPS


@jax.jit
def _stage_c1(h2, mu2, inv2, g, b, x0):
    bn = jnp.stack([mu2, inv2, g, b], axis=0)
    return pl.pallas_call(
        _stage_c1_body,
        grid=(GRID,),
        in_specs=[
            pl.BlockSpec((BN_BLK, D), lambda i: (i, 0)),
            pl.BlockSpec((4, D), lambda i: (0, 0)),
            pl.BlockSpec((BN_BLK, D), lambda i: (i, 0)),
        ],
        out_specs=pl.BlockSpec((BN_BLK, D), lambda i: (i, 0)),
        out_shape=jax.ShapeDtypeStruct((N, D), jnp.float32),
    )(h2, bn, x0)


def _finalize_stats(st):
    mu = st[0] / N
    var = st[1] / N - mu * mu
    return mu, lax.rsqrt(var + 1e-5)


def kernel(x, edge_index, params):
    src = edge_index[0]
    dst = edge_index[1]
    sidx = jnp.concatenate([src, src + N])
    convs = params["convs"]

    v = x
    ep = _prep(v, convs[0]["t"]).reshape(2 * N, D)
    for i in range(2):
        p = convs[i]
        acc = _sc_agg(ep, sidx, dst)
        h1, st1 = _stage_a(acc.reshape(2, N, D), v, p["W1"].T, p["b1"],
                           p["scale"])
        mu1, inv1 = _finalize_stats(st1)
        h2, st2 = _stage_b(h1, mu1, inv1, p["g1"], p["be1"], p["W2"].T,
                           p["b2"])
        mu2, inv2 = _finalize_stats(st2)
        if i == 0:
            v, ep2 = _stage_c0(h2, mu2, inv2, params["norm_g"][0],
                               params["norm_b"][0], convs[1]["t"])
            ep = ep2.reshape(2 * N, D)
        else:
            out = _stage_c1(h2, mu2, inv2, params["norm_g"][1],
                            params["norm_b"][1], x)
    return out


# speedup vs baseline: 1.1084x; 1.0480x over previous
import functools

import jax
import jax.numpy as jnp
from jax import lax
from jax.experimental import pallas as pl
from jax.experimental.pallas import tpu as pltpu
from jax.experimental.pallas import tpu_sc as plsc

N = 10000
E = 320000
D = 128
H = D // 2
D2 = 2 * D
EPS = 1e-7

NC = 2
NS = 16
CH = 104
EPT = E // NS
NCH = EPT // CH
TAIL = EPT - NCH * CH
RPT = (N // NS) // 8 * 8
RTAIL = N - RPT * NS

BN_BLK = 2000
GRID = N // BN_BLK


def _sc_body(ep_hbm, si_hbm, di_hbm, out_hbm, acc,
             ia0, ia1, ia2, id0, id1, id2, bf0, bf1, bf2,
             ibt, idt,
             is0, is1, is2, gs0, gs1, gs2, tsem):
    c = lax.axis_index("c")
    s = lax.axis_index("s")
    base = s * EPT
    ibs = [ia0, ia1, ia2]
    ibd = [id0, id1, id2]
    buf = [bf0, bf1, bf2]
    isem = [is0, is1, is2]
    gsem = [gs0, gs1, gs2]

    def idx_start(chunk, slot):
        off = base + chunk * CH
        pltpu.async_copy(si_hbm.at[pl.ds(c * E + off, CH)], ibs[slot],
                         isem[slot])
        pltpu.async_copy(di_hbm.at[pl.ds(off, CH)], ibd[slot], isem[slot])

    def idx_wait(slot):
        pltpu.make_async_copy(si_hbm.at[pl.ds(0, CH)], ibs[slot],
                              isem[slot]).wait()
        pltpu.make_async_copy(di_hbm.at[pl.ds(0, CH)], ibd[slot],
                              isem[slot]).wait()

    def gather_start(slot):
        pltpu.async_copy(ep_hbm.at[ibs[slot]], buf[slot], gsem[slot])

    def gather_wait(slot):
        pltpu.make_async_copy(ep_hbm.at[ibs[slot]], buf[slot],
                              gsem[slot]).wait()

    def scatter_sync(slot):
        pltpu.sync_copy(buf[slot], acc.at[ibd[slot]], add=True)

    idx_start(0, 0)
    idx_start(1, 1)
    idx_start(2, 2)

    def _z(i, _):
        bf2[i // 8, pl.ds((i % 8) * 16, 16)] = jnp.zeros((16,), jnp.float32)
        return 0
    lax.fori_loop(0, CH * 8, _z, 0)

    def _zcp(j, _):
        pltpu.sync_copy(bf2, acc.at[pl.ds(s * RPT + j * CH, CH)])
        return 0
    lax.fori_loop(0, RPT // CH, _zcp, 0)
    if RTAIL:
        @pl.when(s == 0)
        def _():
            pltpu.sync_copy(bf2.at[pl.ds(0, RTAIL)],
                            acc.at[pl.ds(NS * RPT, RTAIL)])

    idx_wait(0)
    gather_start(0)
    idx_wait(1)
    gather_start(1)
    plsc.subcore_barrier()

    def _chunk_body(chk, slot, prefetch):
        gather_wait(slot)
        scatter_sync(slot)
        if prefetch:
            idx_start(chk + 3, slot)
            idx_wait((slot + 2) % 3)
            gather_start((slot + 2) % 3)

    def _loop(m, _):
        for k in range(3):
            _chunk_body(3 * m + k, k, True)
        return 0
    lax.fori_loop(0, (NCH - 3) // 3, _loop, 0)

    _chunk_body(NCH - 3, 0, False)
    idx_wait(2)
    gather_start(2)
    _chunk_body(NCH - 2, 1, False)
    _chunk_body(NCH - 1, 2, False)

    if TAIL:
        toff = base + NCH * CH
        pltpu.sync_copy(si_hbm.at[pl.ds(c * E + toff, TAIL)], ibt)
        pltpu.sync_copy(di_hbm.at[pl.ds(toff, TAIL)], idt)
        pltpu.async_copy(ep_hbm.at[ibt], bf0.at[pl.ds(0, TAIL)], tsem).wait()
        pltpu.sync_copy(bf0.at[pl.ds(0, TAIL)], acc.at[idt], add=True)

    plsc.subcore_barrier()
    pltpu.sync_copy(acc.at[pl.ds(s * RPT, RPT)],
                    out_hbm.at[pl.ds(c * N + s * RPT, RPT)])
    if RTAIL:
        @pl.when(s == 0)
        def _():
            pltpu.sync_copy(acc.at[pl.ds(NS * RPT, RTAIL)],
                            out_hbm.at[pl.ds(c * N + NS * RPT, RTAIL)])


@jax.jit
def _sc_agg(ep_flat, sidx, didx):
    mesh = plsc.VectorSubcoreMesh(core_axis_name="c", subcore_axis_name="s")
    f = pl.kernel(
        _sc_body,
        out_type=jax.ShapeDtypeStruct((2 * N, D), jnp.float32),
        mesh=mesh,
        scratch_types=(
            [pltpu.VMEM_SHARED((N, D), jnp.float32)]
            + [pltpu.VMEM((CH,), jnp.int32)] * 6
            + [pltpu.VMEM((CH, D), jnp.float32)] * 3
            + [pltpu.VMEM((max(TAIL, 16),), jnp.int32)] * 2
            + [pltpu.SemaphoreType.DMA] * 7
        ),
    )
    return f(ep_flat, sidx, didx)


def _prep_body(v_ref, t_ref, ep_ref):
    m = jnp.maximum(v_ref[...], 0.0) + EPS
    e = jnp.exp(m * t_ref[0, 0])
    p = m * e
    ep_ref[0] = jnp.concatenate([e[:, :H], p[:, :H]], axis=1)
    ep_ref[1] = jnp.concatenate([e[:, H:], p[:, H:]], axis=1)


@jax.jit
def _prep(v, t):
    return pl.pallas_call(
        _prep_body,
        grid=(GRID,),
        in_specs=[
            pl.BlockSpec((BN_BLK, D), lambda i: (i, 0)),
            pl.BlockSpec(memory_space=pltpu.SMEM),
        ],
        out_specs=pl.BlockSpec((2, BN_BLK, D), lambda i: (0, i, 0)),
        out_shape=jax.ShapeDtypeStruct((2, N, D), jnp.float32),
    )(v, t.reshape(1, 1))


def _stage_a_body(acc_ref, v_ref, w1t_ref, b1_ref, sc_ref, h1_ref, st_ref):
    i = pl.program_id(0)
    den = jnp.concatenate([acc_ref[0, :, :H], acc_ref[1, :, :H]], axis=1)
    num = jnp.concatenate([acc_ref[0, :, H:], acc_ref[1, :, H:]], axis=1)
    out = num / (den + 1e-16)
    v = v_ref[...]
    nrm = jnp.maximum(
        jnp.sqrt(jnp.sum(out * out, axis=1, keepdims=True)), 1e-12)
    xn = jnp.sqrt(jnp.sum(v * v, axis=1, keepdims=True))
    z = out * (xn / nrm * sc_ref[0, 0]) + v
    h1 = jnp.dot(z, w1t_ref[...], preferred_element_type=jnp.float32) \
        + b1_ref[...]
    h1_ref[...] = h1
    st = jnp.concatenate([jnp.sum(h1, axis=0, keepdims=True),
                          jnp.sum(h1 * h1, axis=0, keepdims=True)], axis=0)

    @pl.when(i == 0)
    def _():
        st_ref[...] = st

    @pl.when(i > 0)
    def _():
        st_ref[...] = st_ref[...] + st


@jax.jit
def _stage_a(acc, v, w1t, b1, scale):
    return pl.pallas_call(
        _stage_a_body,
        grid=(GRID,),
        in_specs=[
            pl.BlockSpec((2, BN_BLK, D), lambda i: (0, i, 0)),
            pl.BlockSpec((BN_BLK, D), lambda i: (i, 0)),
            pl.BlockSpec((D, D2), lambda i: (0, 0)),
            pl.BlockSpec((1, D2), lambda i: (0, 0)),
            pl.BlockSpec(memory_space=pltpu.SMEM),
        ],
        out_specs=[
            pl.BlockSpec((BN_BLK, D2), lambda i: (i, 0)),
            pl.BlockSpec((2, D2), lambda i: (0, 0)),
        ],
        out_shape=[
            jax.ShapeDtypeStruct((N, D2), jnp.float32),
            jax.ShapeDtypeStruct((2, D2), jnp.float32),
        ],
    )(acc, v, w1t, b1.reshape(1, D2), scale.reshape(1, 1))


def _stage_b_body(h1_ref, bn_ref, w2t_ref, b2_ref, h2_ref, st_ref):
    i = pl.program_id(0)
    a = (h1_ref[...] - bn_ref[0:1, :]) * bn_ref[1:2, :] * bn_ref[2:3, :] \
        + bn_ref[3:4, :]
    a = jnp.maximum(a, 0.0)
    h2 = jnp.dot(a, w2t_ref[...], preferred_element_type=jnp.float32) \
        + b2_ref[...]
    h2_ref[...] = h2
    st = jnp.concatenate([jnp.sum(h2, axis=0, keepdims=True),
                          jnp.sum(h2 * h2, axis=0, keepdims=True)], axis=0)

    @pl.when(i == 0)
    def _():
        st_ref[...] = st

    @pl.when(i > 0)
    def _():
        st_ref[...] = st_ref[...] + st


@jax.jit
def _stage_b(h1, mu1, inv1, g1, be1, w2t, b2):
    bn = jnp.stack([mu1, inv1, g1, be1], axis=0)
    return pl.pallas_call(
        _stage_b_body,
        grid=(GRID,),
        in_specs=[
            pl.BlockSpec((BN_BLK, D2), lambda i: (i, 0)),
            pl.BlockSpec((4, D2), lambda i: (0, 0)),
            pl.BlockSpec((D2, D), lambda i: (0, 0)),
            pl.BlockSpec((1, D), lambda i: (0, 0)),
        ],
        out_specs=[
            pl.BlockSpec((BN_BLK, D), lambda i: (i, 0)),
            pl.BlockSpec((2, D), lambda i: (0, 0)),
        ],
        out_shape=[
            jax.ShapeDtypeStruct((N, D), jnp.float32),
            jax.ShapeDtypeStruct((2, D), jnp.float32),
        ],
    )(h1, bn, w2t, b2.reshape(1, D))


def _stage_c0_body(h2_ref, bn_ref, t_ref, x2_ref, ep_ref):
    y = (h2_ref[...] - bn_ref[0:1, :]) * bn_ref[1:2, :] * bn_ref[2:3, :] \
        + bn_ref[3:4, :]
    x2 = jnp.maximum(y, 0.0) + EPS
    x2_ref[...] = x2
    m = jnp.maximum(x2, 0.0) + EPS
    e = jnp.exp(m * t_ref[0, 0])
    p = m * e
    ep_ref[0] = jnp.concatenate([e[:, :H], p[:, :H]], axis=1)
    ep_ref[1] = jnp.concatenate([e[:, H:], p[:, H:]], axis=1)


@jax.jit
def _stage_c0(h2, mu2, inv2, g, b, t_next):
    bn = jnp.stack([mu2, inv2, g, b], axis=0)
    return pl.pallas_call(
        _stage_c0_body,
        grid=(GRID,),
        in_specs=[
            pl.BlockSpec((BN_BLK, D), lambda i: (i, 0)),
            pl.BlockSpec((4, D), lambda i: (0, 0)),
            pl.BlockSpec(memory_space=pltpu.SMEM),
        ],
        out_specs=[
            pl.BlockSpec((BN_BLK, D), lambda i: (i, 0)),
            pl.BlockSpec((2, BN_BLK, D), lambda i: (0, i, 0)),
        ],
        out_shape=[
            jax.ShapeDtypeStruct((N, D), jnp.float32),
            jax.ShapeDtypeStruct((2, N, D), jnp.float32),
        ],
    )(h2, bn, t_next.reshape(1, 1))


def _stage_c1_body(h2_ref, bn_ref, x_ref, out_ref):
    y = (h2_ref[...] - bn_ref[0:1, :]) * bn_ref[1:2, :] * bn_ref[2:3, :] \
        + bn_ref[3:4, :]
    out_ref[...] = jnp.maximum(x_ref[...] + y, 0.0) + EPS


@jax.jit
def _stage_c1(h2, mu2, inv2, g, b, x0):
    bn = jnp.stack([mu2, inv2, g, b], axis=0)
    return pl.pallas_call(
        _stage_c1_body,
        grid=(GRID,),
        in_specs=[
            pl.BlockSpec((BN_BLK, D), lambda i: (i, 0)),
            pl.BlockSpec((4, D), lambda i: (0, 0)),
            pl.BlockSpec((BN_BLK, D), lambda i: (i, 0)),
        ],
        out_specs=pl.BlockSpec((BN_BLK, D), lambda i: (i, 0)),
        out_shape=jax.ShapeDtypeStruct((N, D), jnp.float32),
    )(h2, bn, x0)


def _finalize_stats(st):
    mu = st[0] / N
    var = st[1] / N - mu * mu
    return mu, lax.rsqrt(var + 1e-5)


def kernel(x, edge_index, params):
    src = edge_index[0]
    dst = edge_index[1]
    sidx = jnp.concatenate([src, src + N])
    convs = params["convs"]

    v = x
    ep = _prep(v, convs[0]["t"]).reshape(2 * N, D)
    for i in range(2):
        p = convs[i]
        acc = _sc_agg(ep, sidx, dst)
        h1, st1 = _stage_a(acc.reshape(2, N, D), v, p["W1"].T, p["b1"],
                           p["scale"])
        mu1, inv1 = _finalize_stats(st1)
        h2, st2 = _stage_b(h1, mu1, inv1, p["g1"], p["be1"], p["W2"].T,
                           p["b2"])
        mu2, inv2 = _finalize_stats(st2)
        if i == 0:
            v, ep2 = _stage_c0(h2, mu2, inv2, params["norm_g"][0],
                               params["norm_b"][0], convs[1]["t"])
            ep = ep2.reshape(2 * N, D)
        else:
            out = _stage_c1(h2, mu2, inv2, params["norm_g"][1],
                            params["norm_b"][1], x)
    return out
